# baseline (speedup 1.0000x reference)
import functools

import jax
import jax.numpy as jnp
from jax.experimental import pallas as pl
from jax.experimental.pallas import tpu as pltpu

_B, _S, _D = 16, 4096, 1024
_BS = 256
_NBJ = _S // _BS


def _tc_body(nblk_ref, lens_ref, hs_ref, o_ref, acc_ref):
    i = pl.program_id(0)
    j = pl.program_id(1)

    @pl.when(j == 0)
    def _():
        acc_ref[...] = jnp.zeros_like(acc_ref)

    L = lens_ref[i]
    active = j < nblk_ref[i]
    full = active & (j > 0) & ((j + 1) * _BS - 1 <= L)

    @pl.when(full)
    def _():
        xs = hs_ref[...].reshape(_BS // 8, 8, _D)
        acc_ref[...] += jnp.sum(xs, axis=0)

    @pl.when(active & jnp.logical_not(full))
    def _():
        pos = jax.lax.broadcasted_iota(jnp.int32, (_BS, 1), 0) + j * _BS
        m = ((pos >= 1) & (pos <= L)).astype(jnp.float32)
        xs = (hs_ref[...] * m).reshape(_BS // 8, 8, _D)
        acc_ref[...] += jnp.sum(xs, axis=0)

    @pl.when(j == _NBJ - 1)
    def _():
        inv = 1.0 / L.astype(jnp.float32)
        o_ref[0] = jnp.sum(acc_ref[...], axis=0, keepdims=True) * inv


@jax.jit
def kernel(hidden_states, pivot_len_list):
    hs2 = hidden_states.reshape(_B * _S, _D)
    nblk = pivot_len_list // _BS + 1
    grid_spec = pltpu.PrefetchScalarGridSpec(
        num_scalar_prefetch=2,
        grid=(_B, _NBJ),
        in_specs=[
            pl.BlockSpec(
                (_BS, _D),
                lambda i, j, nblk_ref, lens_ref: (
                    i * _NBJ + jnp.minimum(j, nblk_ref[i] - 1), 0),
            ),
        ],
        out_specs=pl.BlockSpec(
            (1, 1, _D), lambda i, j, nblk_ref, lens_ref: (i, 0, 0)),
        scratch_shapes=[pltpu.VMEM((8, _D), jnp.float32)],
    )
    pool = pl.pallas_call(
        _tc_body,
        grid_spec=grid_spec,
        out_shape=jax.ShapeDtypeStruct((_B, 1, _D), jnp.float32),
    )
    return pool(nblk, pivot_len_list, hs2).reshape(_B, _D)


# speedup vs baseline: 1.4227x; 1.4227x over previous
import functools

import jax
import jax.numpy as jnp
from jax.experimental import pallas as pl
from jax.experimental.pallas import tpu as pltpu

_B, _S, _D = 16, 4096, 1024
_BS = 1024
_NBJ = _S // _BS


def _tc_body(nblk_ref, lens_ref, hs_ref, o_ref, acc_ref):
    i = pl.program_id(0)
    j = pl.program_id(1)

    @pl.when(j == 0)
    def _():
        acc_ref[...] = jnp.zeros_like(acc_ref)

    L = lens_ref[i]
    active = j < nblk_ref[i]
    full = active & (j > 0) & ((j + 1) * _BS - 1 <= L)

    @pl.when(full)
    def _():
        xs = hs_ref[...].reshape(_BS // 8, 8, _D)
        acc_ref[...] += jnp.sum(xs, axis=0)

    @pl.when(active & jnp.logical_not(full))
    def _():
        pos = jax.lax.broadcasted_iota(jnp.int32, (_BS, 1), 0) + j * _BS
        m = ((pos >= 1) & (pos <= L)).astype(jnp.float32)
        xs = (hs_ref[...] * m).reshape(_BS // 8, 8, _D)
        acc_ref[...] += jnp.sum(xs, axis=0)

    @pl.when(j == _NBJ - 1)
    def _():
        inv = 1.0 / L.astype(jnp.float32)
        o_ref[0] = jnp.sum(acc_ref[...], axis=0, keepdims=True) * inv


@jax.jit
def kernel(hidden_states, pivot_len_list):
    hs2 = hidden_states.reshape(_B * _S, _D)
    nblk = pivot_len_list // _BS + 1
    grid_spec = pltpu.PrefetchScalarGridSpec(
        num_scalar_prefetch=2,
        grid=(_B, _NBJ),
        in_specs=[
            pl.BlockSpec(
                (_BS, _D),
                lambda i, j, nblk_ref, lens_ref: (
                    i * _NBJ + jnp.minimum(j, nblk_ref[i] - 1), 0),
            ),
        ],
        out_specs=pl.BlockSpec(
            (1, 1, _D), lambda i, j, nblk_ref, lens_ref: (i, 0, 0)),
        scratch_shapes=[pltpu.VMEM((8, _D), jnp.float32)],
    )
    pool = pl.pallas_call(
        _tc_body,
        grid_spec=grid_spec,
        out_shape=jax.ShapeDtypeStruct((_B, 1, _D), jnp.float32),
    )
    return pool(nblk, pivot_len_list, hs2).reshape(_B, _D)
